# baseline (speedup 1.0000x reference)
import jax
import jax.numpy as jnp
from jax import lax
from jax.experimental import pallas as pl
from jax.experimental.pallas import tpu as pltpu
from jax.experimental.pallas import tpu_sc as plsc

NUM_BAGS = 26
VOCAB = 100000
DIM = 32
BATCH = 4096
LPS = 50
W = NUM_BAGS * DIM
CREG = W // 16

NC = 2
NS = 16
NW = NC * NS
SPW = BATCH // NW
CS = 16
NCHUNK = SPW // CS


def _body(idx_hbm, tab_hbm, out_hbm, idx_v, rows0, rows1, out_v, sem0, sem1):
    wid = lax.axis_index("s") * NC + lax.axis_index("c")
    base_sample = wid * SPW

    pltpu.sync_copy(idx_hbm.at[pl.ds(base_sample, SPW)], idx_v)

    bufs = (rows0, rows1)
    sems = (sem0, sem1)

    def start(t, parity):
        pltpu.async_copy(tab_hbm.at[idx_v.at[t]], bufs[parity], sems[parity])

    def wait(parity):
        pltpu.make_async_copy(
            tab_hbm.at[idx_v.at[0]], bufs[parity], sems[parity]
        ).wait()

    def reduce(t, buf):
        s_local = t & (CS - 1)

        def c_body(c, _):
            col = c * 16
            a = buf[0, pl.ds(col, 16)]
            b = buf[1, pl.ds(col, 16)]
            cc = buf[2, pl.ds(col, 16)]
            d = buf[3, pl.ds(col, 16)]
            for l in range(4, LPS - 2, 4):
                a = a + buf[l, pl.ds(col, 16)]
                b = b + buf[l + 1, pl.ds(col, 16)]
                cc = cc + buf[l + 2, pl.ds(col, 16)]
                d = d + buf[l + 3, pl.ds(col, 16)]
            a = a + buf[48, pl.ds(col, 16)]
            b = b + buf[49, pl.ds(col, 16)]
            bag = c >> 1
            half = (c & 1) * 16
            out_v[bag, s_local, pl.ds(half, 16)] = (a + b) + (cc + d)
            return 0

        lax.fori_loop(0, CREG, c_body, 0)

    def flush(chunk):
        row0 = base_sample + chunk * CS
        pltpu.sync_copy(out_v, out_hbm.at[:, pl.ds(row0, CS), :])

    start(0, 0)

    def pair_body(p, _):
        t0 = p * 2
        t1 = t0 + 1
        start(t1, 1)
        wait(0)
        reduce(t0, rows0)

        @pl.when(t0 + 2 < SPW)
        def _prefetch0():
            start(t0 + 2, 0)

        wait(1)
        reduce(t1, rows1)

        @pl.when((t1 & (CS - 1)) == CS - 1)
        def _flush():
            flush(t1 >> 4)

        return 0

    lax.fori_loop(0, SPW // 2, pair_body, 0)


@jax.jit
def _run(idx_flat, tab_t):
    mesh = plsc.VectorSubcoreMesh(core_axis_name="c", subcore_axis_name="s")
    return pl.kernel(
        _body,
        out_type=jax.ShapeDtypeStruct((NUM_BAGS, BATCH, DIM), jnp.float32),
        mesh=mesh,
        scratch_types=[
            pltpu.VMEM((SPW, LPS), jnp.int32),
            pltpu.VMEM((LPS, W), jnp.float32),
            pltpu.VMEM((LPS, W), jnp.float32),
            pltpu.VMEM((NUM_BAGS, CS, DIM), jnp.float32),
            pltpu.SemaphoreType.DMA,
            pltpu.SemaphoreType.DMA,
        ],
        compiler_params=pltpu.CompilerParams(use_tc_tiling_on_sc=False),
    )(idx_flat, tab_t)


def kernel(inputs, tables):
    tab_t = jnp.transpose(tables, (1, 0, 2)).reshape(VOCAB, W)
    return _run(inputs, tab_t).reshape(NUM_BAGS * BATCH, DIM)


# speedup vs baseline: 1.0556x; 1.0556x over previous
import jax
import jax.numpy as jnp
from jax import lax
from jax.experimental import pallas as pl
from jax.experimental.pallas import tpu as pltpu
from jax.experimental.pallas import tpu_sc as plsc

NUM_BAGS = 26
VOCAB = 100000
DIM = 32
BATCH = 4096
LPS = 50
W = NUM_BAGS * DIM
CREG = W // 16

NC = 2
NS = 16
NW = NC * NS
SPW = BATCH // NW
CS = 16
NCHUNK = SPW // CS


def _body(idx_hbm, tab_hbm, out_hbm, idx_v, rows0, rows1, out_v, sem0, sem1):
    wid = lax.axis_index("s") * NC + lax.axis_index("c")
    base_sample = wid * SPW

    pltpu.sync_copy(idx_hbm.at[pl.ds(base_sample, SPW)], idx_v)

    bufs = (rows0, rows1)
    sems = (sem0, sem1)

    def start(t, parity):
        pltpu.async_copy(tab_hbm.at[idx_v.at[t]], bufs[parity], sems[parity])

    def wait(parity):
        pltpu.make_async_copy(
            tab_hbm.at[idx_v.at[0]], bufs[parity], sems[parity]
        ).wait()

    def reduce(t, buf):
        s_local = t & (CS - 1)

        def c_body(c, _):
            col = c * 16
            a = buf[0, pl.ds(col, 16)]
            b = buf[1, pl.ds(col, 16)]
            cc = buf[2, pl.ds(col, 16)]
            d = buf[3, pl.ds(col, 16)]
            for l in range(4, LPS - 2, 4):
                a = a + buf[l, pl.ds(col, 16)]
                b = b + buf[l + 1, pl.ds(col, 16)]
                cc = cc + buf[l + 2, pl.ds(col, 16)]
                d = d + buf[l + 3, pl.ds(col, 16)]
            a = a + buf[48, pl.ds(col, 16)]
            b = b + buf[49, pl.ds(col, 16)]
            bag = c >> 1
            half = (c & 1) * 16
            out_v[bag, s_local, pl.ds(half, 16)] = (a + b) + (cc + d)
            return 0

        lax.fori_loop(0, CREG, c_body, 0)

    def flush(chunk):
        row0 = base_sample + chunk * CS
        pltpu.sync_copy(out_v, out_hbm.at[:, pl.ds(row0, CS), :])

    start(0, 0)

    def pair_body(p, _):
        t0 = p * 2
        t1 = t0 + 1
        start(t1, 1)
        wait(0)
        reduce(t0, rows0)

        @pl.when(t0 + 2 < SPW)
        def _prefetch0():
            start(t0 + 2, 0)

        wait(1)
        reduce(t1, rows1)

        @pl.when((t1 & (CS - 1)) == CS - 1)
        def _flush():
            flush(t1 >> 4)

        return 0

    lax.fori_loop(0, SPW // 2, pair_body, 0)


VC = 1000


def _transpose_body(x_ref, o_ref):
    x = x_ref[...]
    o_ref[...] = jnp.concatenate([x[b] for b in range(NUM_BAGS)], axis=1)


def _transpose_tc(tables):
    return pl.pallas_call(
        _transpose_body,
        grid=(VOCAB // VC,),
        in_specs=[pl.BlockSpec((NUM_BAGS, VC, DIM), lambda i: (0, i, 0))],
        out_specs=pl.BlockSpec((VC, W), lambda i: (i, 0)),
        out_shape=jax.ShapeDtypeStruct((VOCAB, W), jnp.float32),
    )(tables)


@jax.jit
def _run(idx_flat, tables):
    tab_t = _transpose_tc(tables)
    mesh = plsc.VectorSubcoreMesh(core_axis_name="c", subcore_axis_name="s")
    return pl.kernel(
        _body,
        out_type=jax.ShapeDtypeStruct((NUM_BAGS, BATCH, DIM), jnp.float32),
        mesh=mesh,
        scratch_types=[
            pltpu.VMEM((SPW, LPS), jnp.int32),
            pltpu.VMEM((LPS, W), jnp.float32),
            pltpu.VMEM((LPS, W), jnp.float32),
            pltpu.VMEM((NUM_BAGS, CS, DIM), jnp.float32),
            pltpu.SemaphoreType.DMA,
            pltpu.SemaphoreType.DMA,
        ],
        compiler_params=pltpu.CompilerParams(use_tc_tiling_on_sc=False),
    )(idx_flat, tab_t)


def kernel(inputs, tables):
    return _run(inputs, tables).reshape(NUM_BAGS * BATCH, DIM)


# speedup vs baseline: 1.4332x; 1.3578x over previous
import jax
import jax.numpy as jnp
from jax import lax
from jax.experimental import pallas as pl
from jax.experimental.pallas import tpu as pltpu
from jax.experimental.pallas import tpu_sc as plsc

NUM_BAGS = 26
VOCAB = 100000
DIM = 32
BATCH = 4096
LPS = 50

NC = 2
NS = 16
NW = NC * NS
SPW = BATCH // NW
S = 8
SB = SPW // S
SB_SHIFT = SB.bit_length() - 1
ROWS = S * LPS
T = NUM_BAGS * SB
NBUF = 4


def _body(idx_hbm, tab_hbm, out_hbm, idx_v, b0, b1, b2, b3, out_v,
          s0, s1, s2, s3):
    wid = lax.axis_index("s") * NC + lax.axis_index("c")
    base_sample = wid * SPW

    bufs = (b0, b1, b2, b3)
    sems = (s0, s1, s2, s3)

    pltpu.sync_copy(idx_hbm.at[pl.ds(base_sample * LPS, SPW * LPS)], idx_v)

    def start(t, j):
        bag = t >> SB_SHIFT
        sb = t & (SB - 1)
        idx_slice = idx_v.at[pl.ds(sb * ROWS, ROWS)]
        pltpu.async_copy(tab_hbm.at[bag].at[idx_slice], bufs[j], sems[j])

    def wait(j):
        pltpu.make_async_copy(
            tab_hbm.at[0].at[idx_v.at[pl.ds(0, ROWS)]], bufs[j], sems[j]
        ).wait()

    def reduce(t, buf):
        sb = t & (SB - 1)

        def s_body(s2_, _):
            for u in range(2):
                s = s2_ * 2 + u
                r0 = s * LPS
                a0 = buf[r0, 0:16]
                a1 = buf[r0, 16:32]
                c0 = buf[r0 + 1, 0:16]
                c1 = buf[r0 + 1, 16:32]
                for l in range(2, LPS, 2):
                    a0 = a0 + buf[r0 + l, 0:16]
                    a1 = a1 + buf[r0 + l, 16:32]
                for l in range(3, LPS, 2):
                    c0 = c0 + buf[r0 + l, 0:16]
                    c1 = c1 + buf[r0 + l, 16:32]
                row = sb * S + s
                out_v[row, 0:16] = a0 + c0
                out_v[row, 16:32] = a1 + c1
            return 0

        lax.fori_loop(0, S // 2, s_body, 0)

    for j in range(NBUF):
        start(j, j)

    def quad_body(q, _):
        tq = q * NBUF
        for j in range(NBUF):
            t = tq + j
            wait(j)
            reduce(t, bufs[j])

            @pl.when(t + NBUF < T)
            def _prefetch():
                start(t + NBUF, j)

            if j == NBUF - 1:
                @pl.when((t & (SB - 1)) == SB - 1)
                def _flush():
                    bag = t >> SB_SHIFT
                    pltpu.sync_copy(
                        out_v,
                        out_hbm.at[pl.ds(bag * BATCH + base_sample, SPW)],
                    )

        return 0

    lax.fori_loop(0, T // NBUF, quad_body, 0)


@jax.jit
def _run(idx_flat, tables):
    mesh = plsc.VectorSubcoreMesh(core_axis_name="c", subcore_axis_name="s")
    return pl.kernel(
        _body,
        out_type=jax.ShapeDtypeStruct((NUM_BAGS * BATCH, DIM), jnp.float32),
        mesh=mesh,
        scratch_types=[
            pltpu.VMEM((SPW * LPS,), jnp.int32),
            pltpu.VMEM((ROWS, DIM), jnp.float32),
            pltpu.VMEM((ROWS, DIM), jnp.float32),
            pltpu.VMEM((ROWS, DIM), jnp.float32),
            pltpu.VMEM((ROWS, DIM), jnp.float32),
            pltpu.VMEM((SPW, DIM), jnp.float32),
            pltpu.SemaphoreType.DMA,
            pltpu.SemaphoreType.DMA,
            pltpu.SemaphoreType.DMA,
            pltpu.SemaphoreType.DMA,
        ],
        compiler_params=pltpu.CompilerParams(use_tc_tiling_on_sc=False),
    )(idx_flat, tables)


def kernel(inputs, tables):
    return _run(inputs.reshape(-1), tables)
